# Initial kernel scaffold; baseline (speedup 1.0000x reference)
import jax
import jax.numpy as jnp
from jax.experimental import pallas as pl


def kernel(x, edge_index, W1l, b1l, W1r, W2l, b2l, W2r, Wo, bo):
    raise NotImplementedError("write your pallas kernel here")


# speedup vs baseline: 5.2421x; 5.2421x over previous
import jax
import jax.numpy as jnp
from jax import lax
from jax.experimental import pallas as pl
from jax.experimental.pallas import tpu as pltpu
from jax.experimental.pallas import tpu_sc as plsc

_N = 10000
_NPAD = 10240
_D = 128
_E = 320000
_NC = 2
_NS = 16
_NW = _NC * _NS
_EPT = _E // _NW
_CHUNK = 80
_NCHUNK = _EPT // _CHUNK
_RPT = _NPAD // _NS
_ZR = 128
_NDUMP = _RPT // _ZR
_BLK = 256


def _sc_agg_body(with_counts, *refs):
    if with_counts:
        (table, src, dst, agg_out, cnt_out,
         src_v, dst_v, rows_v, buf_v, cnt_v, agg_sh, sem) = refs
    else:
        (table, src, dst, agg_out,
         src_v, dst_v, rows_v, buf_v, agg_sh, sem) = refs
    c = lax.axis_index("c")
    s = lax.axis_index("s")
    zeros16 = jnp.zeros((16,), jnp.float32)

    def _zb(i, carry):
        r = i // (_D // 16)
        col = (i % (_D // 16)) * 16
        buf_v[r, pl.ds(col, 16)] = zeros16
        return carry
    lax.fori_loop(0, _ZR * _D // 16, _zb, 0)
    base = s * _RPT
    for r in range(_NDUMP):
        pltpu.sync_copy(buf_v, agg_sh.at[pl.ds(base + r * _ZR, _ZR)])
    if with_counts:
        def _zc(i, carry):
            cnt_v[pl.ds(i * 16, 16)] = zeros16
            return carry
        lax.fori_loop(0, _NPAD // 16, _zc, 0)
    plsc.subcore_barrier()

    ebase = (s * _NC + c) * _EPT
    ones16 = jnp.ones((16,), jnp.float32)

    def _chunk(g, carry):
        b = ebase + g * _CHUNK
        pltpu.sync_copy(src.at[pl.ds(b, _CHUNK)], src_v)
        pltpu.sync_copy(dst.at[pl.ds(b, _CHUNK)], dst_v)
        pltpu.async_copy(table.at[src_v], rows_v, sem).wait()
        pltpu.sync_copy(rows_v, agg_sh.at[dst_v], add=True)
        if with_counts:
            for j in range(_CHUNK // 16):
                idx = dst_v[pl.ds(j * 16, 16)]
                plsc.addupdate_scatter(cnt_v, [idx], ones16)
        return carry
    lax.fori_loop(0, _NCHUNK, _chunk, 0)
    plsc.subcore_barrier()

    for r in range(_NDUMP):
        pltpu.sync_copy(agg_sh.at[pl.ds(base + r * _ZR, _ZR)], buf_v)
        pltpu.sync_copy(buf_v, agg_out.at[c, pl.ds(base + r * _ZR, _ZR)])
    if with_counts:
        pltpu.sync_copy(cnt_v, cnt_out.at[c, s])


def _make_sc_agg(with_counts):
    mesh = plsc.VectorSubcoreMesh(core_axis_name="c", subcore_axis_name="s",
                                  num_cores=_NC, num_subcores=_NS)
    out_type = [jax.ShapeDtypeStruct((_NC, _NPAD, _D), jnp.float32)]
    if with_counts:
        out_type.append(jax.ShapeDtypeStruct((_NC, _NS, _NPAD), jnp.float32))
    scratch = [
        pltpu.VMEM((_CHUNK,), jnp.int32),
        pltpu.VMEM((_CHUNK,), jnp.int32),
        pltpu.VMEM((_CHUNK, _D), jnp.float32),
        pltpu.VMEM((_ZR, _D), jnp.float32),
    ]
    if with_counts:
        scratch.append(pltpu.VMEM((_NPAD,), jnp.float32))
    scratch.append(pltpu.VMEM_SHARED((_NPAD, _D), jnp.float32))
    scratch.append(pltpu.SemaphoreType.DMA)

    def body(*refs):
        _sc_agg_body(with_counts, *refs)
    return pl.kernel(body, out_type=tuple(out_type), mesh=mesh,
                     compiler_params=pltpu.CompilerParams(needs_layout_passes=False),
                     scratch_types=tuple(scratch))


_SC_CACHE = {}


def _sc_agg(with_counts, *args):
    if with_counts not in _SC_CACHE:
        _SC_CACHE[with_counts] = _make_sc_agg(with_counts)
    return _SC_CACHE[with_counts](*args)


def _tc_layer1_body(agg_ref, cnt_ref, x_ref, wl_ref, wr_ref, b_ref, out_ref):
    agg = agg_ref[0] + agg_ref[1]
    cnt = jnp.sum(cnt_ref[...].reshape(_NC * _NS, _BLK), axis=0)
    inv = 1.0 / jnp.maximum(cnt, 1.0)
    mean = agg * inv[:, None]
    h = lax.dot_general(mean, wl_ref[...], (((1,), (1,)), ((), ())),
                        preferred_element_type=jnp.float32)
    h = h + lax.dot_general(x_ref[...], wr_ref[...], (((1,), (1,)), ((), ())),
                            preferred_element_type=jnp.float32)
    h = h + b_ref[...]
    out_ref[...] = jnp.maximum(h, 0.0)


def _tc_layer2_body(agg_ref, cnt_ref, x_ref, wl_ref, wr_ref, b_ref,
                    wo_ref, bo_ref, h_ref, logit_ref):
    agg = agg_ref[0] + agg_ref[1]
    cnt = jnp.sum(cnt_ref[...].reshape(_NC * _NS, _BLK), axis=0)
    inv = 1.0 / jnp.maximum(cnt, 1.0)
    mean = agg * inv[:, None]
    h = lax.dot_general(mean, wl_ref[...], (((1,), (1,)), ((), ())),
                        preferred_element_type=jnp.float32)
    h = h + lax.dot_general(x_ref[...], wr_ref[...], (((1,), (1,)), ((), ())),
                            preferred_element_type=jnp.float32)
    h = h + b_ref[...]
    h = jnp.maximum(h, 0.0)
    h_ref[...] = h
    logit_ref[...] = lax.dot_general(h, wo_ref[...], (((1,), (1,)), ((), ())),
                                     preferred_element_type=jnp.float32) + bo_ref[...]


_agg_spec = pl.BlockSpec((_NC, _BLK, _D), lambda i: (0, i, 0))
_cnt_spec = pl.BlockSpec((_NC, _NS, _BLK), lambda i: (0, 0, i))
_row_spec = pl.BlockSpec((_BLK, _D), lambda i: (i, 0))
_w_spec = pl.BlockSpec((_D, _D), lambda i: (0, 0))
_b_spec = pl.BlockSpec((1, _D), lambda i: (0, 0))


def _tc_layer1(agg, cnt, x, wl, wr, b):
    return pl.pallas_call(
        _tc_layer1_body,
        grid=(_NPAD // _BLK,),
        in_specs=[_agg_spec, _cnt_spec, _row_spec, _w_spec, _w_spec, _b_spec],
        out_specs=_row_spec,
        out_shape=jax.ShapeDtypeStruct((_NPAD, _D), jnp.float32),
    )(agg, cnt, x, wl, wr, b)


def _tc_layer2(agg, cnt, h1, wl, wr, b, wo, bo):
    return pl.pallas_call(
        _tc_layer2_body,
        grid=(_NPAD // _BLK,),
        in_specs=[_agg_spec, _cnt_spec, _row_spec, _w_spec, _w_spec, _b_spec,
                  _w_spec, _b_spec],
        out_specs=(_row_spec, _row_spec),
        out_shape=(jax.ShapeDtypeStruct((_NPAD, _D), jnp.float32),
                   jax.ShapeDtypeStruct((_NPAD, _D), jnp.float32)),
    )(agg, cnt, h1, wl, wr, b, wo, bo)


def kernel(x, edge_index, W1l, b1l, W1r, W2l, b2l, W2r, Wo, bo):
    src = edge_index[0]
    dst = edge_index[1]
    xp = jnp.zeros((_NPAD, _D), jnp.float32).at[:_N].set(x)
    agg1, cnt = _sc_agg(True, xp, src, dst)
    h1 = _tc_layer1(agg1, cnt, xp, W1l, W1r, b1l.reshape(1, _D))
    (agg2,) = _sc_agg(False, h1, src, dst)
    wo_pad = jnp.zeros((_D, _D), jnp.float32).at[:Wo.shape[0]].set(Wo)
    bo_pad = jnp.zeros((1, _D), jnp.float32).at[0, :bo.shape[0]].set(bo)
    h2, logits_pad = _tc_layer2(agg2, cnt, h1, W2l, W2r, b2l.reshape(1, _D),
                                wo_pad, bo_pad)
    return (logits_pad[:_N, :Wo.shape[0]], h2[:_N])
